# baseline (speedup 1.0000x reference)
import jax
import jax.numpy as jnp
from jax.experimental import pallas as pl

MAX_POSITIONS = 8192
SEQ_BLOCK = 256


def _add_kernel(x_ref, emb_ref, out_ref):
    out_ref[...] = x_ref[...] + emb_ref[...][None, :, :]


def kernel(x, emb_weight):
    batch, seq_len, dim = x.shape
    offset_blocks = (emb_weight.shape[0] - seq_len) // SEQ_BLOCK
    num_blocks = seq_len // SEQ_BLOCK
    return pl.pallas_call(
        _add_kernel,
        grid=(num_blocks,),
        in_specs=[
            pl.BlockSpec((batch, SEQ_BLOCK, dim), lambda s: (0, s, 0)),
            pl.BlockSpec((SEQ_BLOCK, dim), lambda s: (s + offset_blocks, 0)),
        ],
        out_specs=pl.BlockSpec((batch, SEQ_BLOCK, dim), lambda s: (0, s, 0)),
        out_shape=jax.ShapeDtypeStruct(x.shape, x.dtype),
    )(x, emb_weight)


# speedup vs baseline: 1.0116x; 1.0116x over previous
import jax
import jax.numpy as jnp
from jax.experimental import pallas as pl
from jax.experimental.pallas import tpu as pltpu

CHUNK = 256
DEPTH = 4


def _pipelined_kernel(x_hbm, emb_hbm, out_hbm, xbuf, ebuf, obuf, xsem, esem, osem):
    batch, seq_len, dim = x_hbm.shape
    off = emb_hbm.shape[0] - seq_len
    n = seq_len // CHUNK

    def x_copy(i, slot):
        return pltpu.make_async_copy(
            x_hbm.at[:, pl.ds(i * CHUNK, CHUNK), :], xbuf.at[slot], xsem.at[slot])

    def e_copy(i, slot):
        return pltpu.make_async_copy(
            emb_hbm.at[pl.ds(off + i * CHUNK, CHUNK), :], ebuf.at[slot], esem.at[slot])

    def o_copy(i, slot):
        return pltpu.make_async_copy(
            obuf.at[slot], out_hbm.at[:, pl.ds(i * CHUNK, CHUNK), :], osem.at[slot])

    for s in range(DEPTH):
        x_copy(s, s).start()
        e_copy(s, s).start()

    def body(i, carry):
        slot = jax.lax.rem(i, DEPTH)
        x_copy(i, slot).wait()
        e_copy(i, slot).wait()

        @pl.when(i >= DEPTH)
        def _():
            o_copy(i - DEPTH, slot).wait()

        obuf[slot] = xbuf[slot] + ebuf[slot][None, :, :]
        o_copy(i, slot).start()

        @pl.when(i + DEPTH < n)
        def _():
            x_copy(i + DEPTH, slot).start()
            e_copy(i + DEPTH, slot).start()

        return carry

    jax.lax.fori_loop(0, n, body, 0)

    for k in range(max(0, n - DEPTH), n):
        o_copy(k, k % DEPTH).wait()


def kernel(x, emb_weight):
    batch, seq_len, dim = x.shape
    return pl.pallas_call(
        _pipelined_kernel,
        in_specs=[
            pl.BlockSpec(memory_space=pltpu.MemorySpace.HBM),
            pl.BlockSpec(memory_space=pltpu.MemorySpace.HBM),
        ],
        out_specs=pl.BlockSpec(memory_space=pltpu.MemorySpace.HBM),
        out_shape=jax.ShapeDtypeStruct(x.shape, x.dtype),
        scratch_shapes=[
            pltpu.MemorySpace.VMEM((DEPTH, batch, CHUNK, dim), jnp.float32),
            pltpu.MemorySpace.VMEM((DEPTH, CHUNK, dim), jnp.float32),
            pltpu.MemorySpace.VMEM((DEPTH, batch, CHUNK, dim), jnp.float32),
            pltpu.SemaphoreType.DMA((DEPTH,)),
            pltpu.SemaphoreType.DMA((DEPTH,)),
            pltpu.SemaphoreType.DMA((DEPTH,)),
        ],
    )(x, emb_weight)
